# baseline (speedup 1.0000x reference)
import functools

import jax
import jax.numpy as jnp
from jax.experimental import pallas as pl
from jax.experimental.pallas import tpu as pltpu

N = 512
B = 8
T_IN = 12
T_OUT = 12
F_IN = 2
H = 32
TG = 4
NG = B * (T_IN // TG)
PLANE = 40
BN_EPS = 1e-5

_HIGHEST = jax.lax.Precision.HIGHEST


def _tgcn_kernel(xp_ref, xg_ref, at_ref, gamma_ref, beta_ref,
                 w1t_ref, b1t_ref, w2dt_ref, b2t_ref,
                 wall_ref, wlin_ref, blin_ref,
                 out_ref, buf):
    f32 = jnp.float32

    xp = xp_ref[...]
    m = jnp.mean(xp, axis=0, keepdims=True)
    xc = xp - m
    v = jnp.mean(xc * xc, axis=0, keepdims=True)
    s = gamma_ref[...] * jax.lax.rsqrt(v + BN_EPS)
    c = beta_ref[...] - s * m

    at = at_ref[...]
    w1t = w1t_ref[...]
    w2dt = w2dt_ref[...]
    b1t = b1t_ref[...]
    b2t = b2t_ref[...]

    for g in range(NG):
        b, j = g // 3, g % 3
        xg = xg_ref[g]
        bn = xg * s + c
        blocks = []
        for i in range(TG):
            blk = (w1t[:, 0:1] * bn[i:i + 1, :]
                   + w1t[:, 1:2] * bn[TG + i:TG + i + 1, :])
            blocks.append(blk)
        y1t = jnp.concatenate(blocks, axis=0)
        t2t = jnp.dot(y1t, at, preferred_element_type=f32,
                      precision=_HIGHEST) + b1t
        t3t = jnp.maximum(t2t, 0.0)
        zt = jnp.dot(w2dt, t3t, preferred_element_type=f32,
                     precision=_HIGHEST)
        t4t = jnp.dot(zt, at, preferred_element_type=f32,
                      precision=_HIGHEST) + b2t
        st = jax.nn.sigmoid(t4t)
        for i in range(TG):
            t = TG * j + i
            r0 = PLANE * (t + 1)
            buf[r0:r0 + H, N * b:N * (b + 1)] = st[H * i:H * (i + 1), :]

    wall = wall_ref[...]
    wlin = wlin_ref[...]
    blin = blin_ref[...]

    const_col = jnp.concatenate(
        [jnp.ones((1, B * N), f32), jnp.zeros((PLANE - H - 1, B * N), f32)],
        axis=0)
    for p in range(T_IN + 1):
        buf[PLANE * p + H:PLANE * (p + 1), :] = const_col
    buf[0:H, :] = jnp.zeros((H, B * N), dtype=f32)
    for k in range(T_OUT + 1):
        n_steps = T_IN if k < T_OUT else 1
        for t in range(n_steps):
            xh = buf[PLANE * t:PLANE * t + 2 * H + 9]
            g = jnp.dot(wall, xh, preferred_element_type=f32)
            rz = jax.nn.sigmoid(g[0:2 * H])
            n = jnp.tanh(g[2 * H:3 * H] + rz[0:H] * g[3 * H:4 * H])
            h = n + rz[H:2 * H] * (xh[0:H] - n)
            if k < T_OUT:
                if t < T_IN - 1 or k < T_OUT - 1:
                    r0 = PLANE * (t + 1)
                    buf[r0:r0 + H, :] = h
                if t == T_IN - 1:
                    buf[0:H, :] = h
            if k >= 1 and t == 0:
                out_ref[k - 1:k, :] = (jnp.sum(h * wlin, axis=0,
                                               keepdims=True) + blin)


@functools.partial(jax.jit, static_argnames=())
def kernel(A, X, bn_gamma, bn_beta, W1, b1, W2, b2,
           W_ih, W_hh, b_ih, b_hh, W_lin, b_lin):
    f32 = jnp.float32
    xpt = jnp.transpose(X, (0, 2, 3, 1)).reshape(B * T_IN * F_IN, N)
    xg = (jnp.transpose(X, (0, 2, 3, 1))
          .reshape(B, T_IN // TG, TG, F_IN, N)
          .transpose(0, 1, 3, 2, 4)
          .reshape(NG, F_IN * TG, N))
    at = A.T
    gamma2 = bn_gamma.reshape(1, N)
    beta2 = bn_beta.reshape(1, N)
    w1t = W1.T
    b1t = jnp.tile(b1, TG).reshape(TG * H, 1)
    w2dt = jnp.kron(jnp.eye(TG, dtype=f32), W2.T)
    b2t = jnp.tile(b2, TG).reshape(TG * H, 1)
    zeros_h = jnp.zeros((H, H), dtype=f32)
    ball = jnp.concatenate([
        (b_ih[0:2 * H] + b_hh[0:2 * H]),
        b_ih[2 * H:3 * H],
        b_hh[2 * H:3 * H],
    ]).reshape(4 * H, 1)
    bhi = ball.astype(jnp.bfloat16).astype(f32)
    blo = ball - bhi
    wh_part = jnp.concatenate([W_hh[0:2 * H], zeros_h,
                               W_hh[2 * H:3 * H]], axis=0)
    wx_part = jnp.concatenate([W_ih[0:2 * H], W_ih[2 * H:3 * H],
                               zeros_h], axis=0)
    pad_cols = jnp.zeros((4 * H, PLANE - H - 1), dtype=f32)
    wall = jnp.concatenate([wh_part, bhi, pad_cols,
                            wx_part, blo], axis=1)
    wlin = W_lin.reshape(H, 1)
    blin = b_lin.reshape(1, 1)

    out = pl.pallas_call(
        _tgcn_kernel,
        out_shape=jax.ShapeDtypeStruct((T_OUT, B * N), f32),
        scratch_shapes=[pltpu.VMEM(((T_IN + 1) * PLANE, B * N), f32)],
    )(xpt, xg, at, gamma2, beta2, w1t, b1t, w2dt, b2t,
      wall, wlin, blin)

    return jnp.transpose(out).reshape(B, N, T_OUT)
